# baseline (speedup 1.0000x reference)
import functools

import jax
import jax.numpy as jnp
from jax import lax
from jax.experimental import pallas as pl
from jax.experimental.pallas import tpu as pltpu
from jax.experimental.pallas import tpu_sc as plsc

L = 16


def _entity_pool_sc(table, idx, masks, M):
    n_rows, D = table.shape
    BE, K = idx.shape
    KH = K // 4
    info = plsc.get_sparse_core_info()
    nw = info.num_cores * info.num_subcores
    epw = BE // nw

    mesh = plsc.VectorSubcoreMesh(core_axis_name="c", subcore_axis_name="s")

    @functools.partial(
        pl.kernel,
        mesh=mesh,
        out_type=jax.ShapeDtypeStruct((BE, D), jnp.float32),
        scratch_types=[
            pltpu.VMEM((epw, K), jnp.int32),
            pltpu.VMEM((epw, K), jnp.int32),
            pltpu.VMEM((KH, D), jnp.float32),
            pltpu.VMEM((KH, D), jnp.float32),
            pltpu.VMEM((KH, D), jnp.float32),
            pltpu.VMEM((KH, D), jnp.float32),
            pltpu.VMEM((KH, D), jnp.float32),
            pltpu.VMEM((KH, D), jnp.float32),
            pltpu.VMEM((KH, D), jnp.float32),
            pltpu.VMEM((KH, D), jnp.float32),
            pltpu.VMEM((epw, D), jnp.float32),
        ] + [pltpu.SemaphoreType.DMA] * 8,
    )
    def run(table_hbm, idx_hbm, mask_hbm, out_hbm,
            idx_v, mask_v, buf0, buf1, buf2, buf3, buf4, buf5, buf6, buf7,
            out_v, sem0, sem1, sem2, sem3, sem4, sem5, sem6, sem7):
        wid = lax.axis_index("s") * info.num_cores + lax.axis_index("c")
        base = wid * epw
        pltpu.sync_copy(idx_hbm.at[pl.ds(base, epw), :], idx_v)
        pltpu.sync_copy(mask_hbm.at[pl.ds(base, epw), :], mask_v)

        boff = jnp.full((L,), (base // (BE // (n_rows // M))) * M,
                        dtype=jnp.int32)
        for e in range(epw):
            for h in range(K // L):
                sl = pl.ds(h * L, L)
                idx_v[e, sl] = idx_v[e, sl] + boff

        bufs = (buf0, buf1, buf2, buf3, buf4, buf5, buf6, buf7)
        sems = (sem0, sem1, sem2, sem3, sem4, sem5, sem6, sem7)

        def slot(e, h):
            return (4 * e + h) % 8

        def copy(e, h):
            s = slot(e, h)
            return pltpu.make_async_copy(
                table_hbm.at[idx_v.at[e, pl.ds(h * KH, KH)]], bufs[s], sems[s])

        for e in (0, 1):
            for h in (0, 1, 2, 3):
                copy(e, h).start()

        for e in range(epw):
            for h in (0, 1, 2, 3):
                copy(e, h).wait()
                buf = bufs[slot(e, h)]
                mv = mask_v[e, pl.ds((h // 2) * L, L)]
                bv = jnp.where(mv == 0, jnp.float32(-1e30), jnp.float32(0.0))
                splats = [jnp.full((L,), bv[(h % 2) * KH + j], dtype=jnp.float32)
                          for j in range(KH)]

                def cbody(c, carry, buf=buf, splats=splats, e=e, h=h):
                    off = c * L
                    if h == 0:
                        acc = buf[0, pl.ds(off, L)] + splats[0]
                        k0 = 1
                    else:
                        acc = out_v[e, pl.ds(off, L)]
                        k0 = 0
                    for kk in range(k0, KH):
                        acc = jnp.maximum(acc, buf[kk, pl.ds(off, L)] + splats[kk])
                    out_v[e, pl.ds(off, L)] = acc
                    return carry

                lax.fori_loop(0, D // L, cbody, 0)
                if e + 2 < epw:
                    copy(e + 2, h).start()
        pltpu.sync_copy(out_v, out_hbm.at[pl.ds(base, epw), :])

    return run(table, idx, masks)


def kernel(mention_reprs, entities, entity_masks):
    B, M, D = mention_reprs.shape
    _, E, K = entities.shape
    table = mention_reprs.reshape(B * M, D)
    idx = entities.reshape(B * E, K)
    masks = entity_masks.reshape(B * E, K)
    out = _entity_pool_sc(table, idx, masks, M)
    return out.reshape(B, E, D)


# speedup vs baseline: 1.1810x; 1.1810x over previous
import functools

import jax
import jax.numpy as jnp
from jax import lax
from jax.experimental import pallas as pl
from jax.experimental.pallas import tpu as pltpu
from jax.experimental.pallas import tpu_sc as plsc

L = 16


def _entity_pool_sc(table, idx, masks, M):
    n_rows, D = table.shape
    BE, K = idx.shape
    KH = K
    info = plsc.get_sparse_core_info()
    nw = info.num_cores * info.num_subcores
    epw = BE // nw

    mesh = plsc.VectorSubcoreMesh(core_axis_name="c", subcore_axis_name="s")

    @functools.partial(
        pl.kernel,
        mesh=mesh,
        out_type=jax.ShapeDtypeStruct((BE, D), jnp.float32),
        scratch_types=[
            pltpu.VMEM((epw, K), jnp.int32),
            pltpu.VMEM((epw, K), jnp.int32),
            pltpu.VMEM((KH, D), jnp.float32),
            pltpu.VMEM((KH, D), jnp.float32),
            pltpu.VMEM((KH, D), jnp.float32),
            pltpu.VMEM((4, D), jnp.float32),
            pltpu.SemaphoreType.DMA,
            pltpu.SemaphoreType.DMA,
            pltpu.SemaphoreType.DMA,
            pltpu.SemaphoreType.DMA,
        ],
    )
    def run(table_hbm, idx_hbm, mask_hbm, out_hbm,
            idx_v, mask_v, buf0, buf1, buf2, out_v,
            sem0, sem1, sem2, osem):
        wid = lax.axis_index("s") * info.num_cores + lax.axis_index("c")
        base = wid * epw
        pltpu.sync_copy(idx_hbm.at[pl.ds(base, epw), :], idx_v)
        pltpu.sync_copy(mask_hbm.at[pl.ds(base, epw), :], mask_v)

        boff = jnp.full((L,), (base // (BE // (n_rows // M))) * M,
                        dtype=jnp.int32)
        for e in range(epw):
            for h in range(K // L):
                sl = pl.ds(h * L, L)
                idx_v[e, sl] = idx_v[e, sl] + boff

        bufs = (buf0, buf1, buf2)
        sems = (sem0, sem1, sem2)

        def copy(e):
            return pltpu.make_async_copy(
                table_hbm.at[idx_v.at[e]], bufs[e % 3], sems[e % 3])

        def ocopy(e):
            return pltpu.make_async_copy(
                out_v.at[pl.ds(e % 4, 1), :],
                out_hbm.at[pl.ds(base + e, 1), :], osem)

        for e in (0, 1, 2):
            copy(e).start()

        for e in range(epw):
            copy(e).wait()
            buf = bufs[e % 3]
            splats = []
            for h in range(K // L):
                mv = mask_v[e, pl.ds(h * L, L)]
                bv = jnp.where(mv == 0, jnp.float32(-1e30), jnp.float32(0.0))
                splats += [jnp.full((L,), bv[j], dtype=jnp.float32)
                           for j in range(L)]
            if e >= 4:
                ocopy(e - 4).wait()

            def cbody(c, carry, buf=buf, splats=splats, e=e):
                off = c * L
                acc = buf[0, pl.ds(off, L)] + splats[0]
                for kk in range(1, K):
                    acc = jnp.maximum(acc, buf[kk, pl.ds(off, L)] + splats[kk])
                out_v[e % 4, pl.ds(off, L)] = acc
                return carry

            lax.fori_loop(0, D // L, cbody, 0)
            ocopy(e).start()
            if e + 3 < epw:
                copy(e + 3).start()
        for e in range(epw - 4, epw):
            ocopy(e).wait()

    return run(table, idx, masks)


def kernel(mention_reprs, entities, entity_masks):
    B, M, D = mention_reprs.shape
    _, E, K = entities.shape
    table = mention_reprs.reshape(B * M, D)
    idx = entities.reshape(B * E, K)
    masks = entity_masks.reshape(B * E, K)
    out = _entity_pool_sc(table, idx, masks, M)
    return out.reshape(B, E, D)


# speedup vs baseline: 1.1898x; 1.0075x over previous
import functools

import jax
import jax.numpy as jnp
from jax import lax
from jax.experimental import pallas as pl
from jax.experimental.pallas import tpu as pltpu
from jax.experimental.pallas import tpu_sc as plsc

L = 16


def _entity_pool_sc(table, idx, masks, M):
    n_rows, D = table.shape
    BE, K = idx.shape
    KH = K // 2
    info = plsc.get_sparse_core_info()
    nw = info.num_cores * info.num_subcores
    epw = BE // nw

    mesh = plsc.VectorSubcoreMesh(core_axis_name="c", subcore_axis_name="s")

    @functools.partial(
        pl.kernel,
        mesh=mesh,
        out_type=jax.ShapeDtypeStruct((BE, D), jnp.float32),
        scratch_types=[
            pltpu.VMEM((epw, K), jnp.int32),
            pltpu.VMEM((epw, K), jnp.int32),
            pltpu.VMEM((KH, D), jnp.float32),
            pltpu.VMEM((KH, D), jnp.float32),
            pltpu.VMEM((KH, D), jnp.float32),
            pltpu.VMEM((KH, D), jnp.float32),
            pltpu.VMEM((epw, D), jnp.float32),
            pltpu.SemaphoreType.DMA,
            pltpu.SemaphoreType.DMA,
            pltpu.SemaphoreType.DMA,
            pltpu.SemaphoreType.DMA,
        ],
    )
    def run(table_hbm, idx_hbm, mask_hbm, out_hbm,
            idx_v, mask_v, buf0, buf1, buf2, buf3, out_v,
            sem0, sem1, sem2, sem3):
        wid = lax.axis_index("s") * info.num_cores + lax.axis_index("c")
        base = wid * epw
        pltpu.sync_copy(idx_hbm.at[pl.ds(base, epw), :], idx_v)
        pltpu.sync_copy(mask_hbm.at[pl.ds(base, epw), :], mask_v)

        boff = jnp.full((L,), (base // (BE // (n_rows // M))) * M,
                        dtype=jnp.int32)
        for e in range(epw):
            for h in range(K // L):
                sl = pl.ds(h * L, L)
                idx_v[e, sl] = idx_v[e, sl] + boff

        bufs = (buf0, buf1, buf2, buf3)
        sems = (sem0, sem1, sem2, sem3)

        def slot(e, h):
            return (2 * e + h) % 4

        def copy(e, h):
            s = slot(e, h)
            return pltpu.make_async_copy(
                table_hbm.at[idx_v.at[e, pl.ds(h * KH, KH)]], bufs[s], sems[s])

        for e in (0, 1):
            for h in (0, 1):
                copy(e, h).start()

        for e in range(epw):
            for h in (0, 1):
                copy(e, h).wait()
                buf = bufs[slot(e, h)]
                mv = mask_v[e, pl.ds(h * KH, L)]
                bv = jnp.where(mv == 0, jnp.float32(-1e30), jnp.float32(0.0))
                splats = [jnp.full((L,), bv[j], dtype=jnp.float32)
                          for j in range(KH)]

                def cbody(c, carry, buf=buf, splats=splats, e=e, h=h):
                    off = c * L
                    if h == 0:
                        acc = buf[0, pl.ds(off, L)] + splats[0]
                        k0 = 1
                    else:
                        acc = out_v[e, pl.ds(off, L)]
                        k0 = 0
                    for kk in range(k0, KH):
                        acc = jnp.maximum(acc, buf[kk, pl.ds(off, L)] + splats[kk])
                    out_v[e, pl.ds(off, L)] = acc
                    return carry

                lax.fori_loop(0, D // L, cbody, 0)
                if e + 2 < epw:
                    copy(e + 2, h).start()
        pltpu.sync_copy(out_v, out_hbm.at[pl.ds(base, epw), :])

    return run(table, idx, masks)


def kernel(mention_reprs, entities, entity_masks):
    B, M, D = mention_reprs.shape
    _, E, K = entities.shape
    table = mention_reprs.reshape(B * M, D)
    idx = entities.reshape(B * E, K)
    masks = entity_masks.reshape(B * E, K)
    out = _entity_pool_sc(table, idx, masks, M)
    return out.reshape(B, E, D)


# speedup vs baseline: 1.2031x; 1.0111x over previous
import functools

import jax
import jax.numpy as jnp
from jax import lax
from jax.experimental import pallas as pl
from jax.experimental.pallas import tpu as pltpu
from jax.experimental.pallas import tpu_sc as plsc

L = 16


def _entity_pool_sc(table, idx, masks, M):
    n_rows, D = table.shape
    BE, K = idx.shape
    KH = K // 2
    info = plsc.get_sparse_core_info()
    nw = info.num_cores * info.num_subcores
    epw = BE // nw

    mesh = plsc.VectorSubcoreMesh(core_axis_name="c", subcore_axis_name="s")

    @functools.partial(
        pl.kernel,
        mesh=mesh,
        out_type=jax.ShapeDtypeStruct((BE, D), jnp.float32),
        scratch_types=[
            pltpu.VMEM((epw, K), jnp.int32),
            pltpu.VMEM((epw, K), jnp.int32),
            pltpu.VMEM((KH, D), jnp.float32),
            pltpu.VMEM((KH, D), jnp.float32),
            pltpu.VMEM((KH, D), jnp.float32),
            pltpu.VMEM((KH, D), jnp.float32),
            pltpu.VMEM((epw, D), jnp.float32),
            pltpu.SemaphoreType.DMA,
            pltpu.SemaphoreType.DMA,
            pltpu.SemaphoreType.DMA,
            pltpu.SemaphoreType.DMA,
            pltpu.SemaphoreType.DMA,
        ],
    )
    def run(table_hbm, idx_hbm, mask_hbm, out_hbm,
            idx_v, mask_v, buf0, buf1, buf2, buf3, out_v,
            sem0, sem1, sem2, sem3, osem):
        wid = lax.axis_index("s") * info.num_cores + lax.axis_index("c")
        base = wid * epw
        pltpu.sync_copy(idx_hbm.at[pl.ds(base, epw), :], idx_v)
        pltpu.sync_copy(mask_hbm.at[pl.ds(base, epw), :], mask_v)

        boff = jnp.full((L,), (base // (BE // (n_rows // M))) * M,
                        dtype=jnp.int32)
        for e in range(epw):
            for h in range(K // L):
                sl = pl.ds(h * L, L)
                idx_v[e, sl] = idx_v[e, sl] + boff

        bufs = (buf0, buf1, buf2, buf3)
        sems = (sem0, sem1, sem2, sem3)

        def slot(e, h):
            return (2 * e + h) % 4

        def copy(e, h):
            s = slot(e, h)
            return pltpu.make_async_copy(
                table_hbm.at[idx_v.at[e, pl.ds(h * KH, KH)]], bufs[s], sems[s])

        for e in (0, 1):
            for h in (0, 1):
                copy(e, h).start()

        for e in range(epw):
            for h in (0, 1):
                copy(e, h).wait()
                buf = bufs[slot(e, h)]
                mv = mask_v[e, pl.ds(h * KH, L)]
                bv = jnp.where(mv == 0, jnp.float32(-1e30), jnp.float32(0.0))
                splats = [jnp.full((L,), bv[j], dtype=jnp.float32)
                          for j in range(KH)]

                def cbody(c, carry, buf=buf, splats=splats, e=e, h=h):
                    off = c * L
                    if h == 0:
                        acc = buf[0, pl.ds(off, L)] + splats[0]
                        k0 = 1
                    else:
                        acc = out_v[e, pl.ds(off, L)]
                        k0 = 0
                    for kk in range(k0, KH):
                        acc = jnp.maximum(acc, buf[kk, pl.ds(off, L)] + splats[kk])
                    out_v[e, pl.ds(off, L)] = acc
                    return carry

                lax.fori_loop(0, D // L, cbody, 0)
                if e + 2 < epw:
                    copy(e + 2, h).start()
            if e % 8 == 7:
                pltpu.make_async_copy(
                    out_v.at[pl.ds(e - 7, 8), :],
                    out_hbm.at[pl.ds(base + e - 7, 8), :], osem).start()
        for e0 in range(0, epw, 8):
            pltpu.make_async_copy(
                out_v.at[pl.ds(e0, 8), :],
                out_hbm.at[pl.ds(base + e0, 8), :], osem).wait()

    return run(table, idx, masks)


def kernel(mention_reprs, entities, entity_masks):
    B, M, D = mention_reprs.shape
    _, E, K = entities.shape
    table = mention_reprs.reshape(B * M, D)
    idx = entities.reshape(B * E, K)
    masks = entity_masks.reshape(B * E, K)
    out = _entity_pool_sc(table, idx, masks, M)
    return out.reshape(B, E, D)
